# baseline (speedup 1.0000x reference)
import jax
import jax.numpy as jnp
from jax.experimental import pallas as pl
from jax.experimental.pallas import tpu as pltpu


def _dense_conv_w(w, hw_in, hw_out, c_in_pad, c_out_pad, row_major_out=False):
    kh, kw, ci, co = w.shape
    off, P = kh // 2, hw_out - 1
    wp = jnp.pad(w, ((P, P), (P, P), (0, c_in_pad - ci), (0, c_out_pad - co)))
    K = hw_in * hw_in * c_in_pad
    rows = []
    for ho in range(hw_out):
        hs = wp[off + P - ho:off + P - ho + hw_in]
        cols = [hs[:, off + P - wo:off + P - wo + hw_in] for wo in range(hw_out)]
        rows.append(jnp.stack(cols, axis=3))
    if row_major_out:
        return jnp.stack(rows, axis=0).reshape(hw_out, K, hw_out * c_out_pad)
    return jnp.stack(rows, axis=3).reshape(K, hw_out * hw_out * c_out_pad)


def _conv1_w(w):
    co = w.shape[-1]
    wt = w.transpose(2, 0, 1, 3)
    blocks = []
    for wo in range(8):
        lo = 4 * wo - 2
        s_lo, s_hi = max(0, -lo), min(7, 32 - lo)
        blk = jnp.pad(wt[:, :, s_lo:s_hi],
                      ((0, 0), (0, 1), (lo + s_lo, 32 - lo - s_hi), (0, 0)))
        blocks.append(blk)
    return jnp.stack(blocks, axis=3).reshape(3, 256, 8 * co)


def _fused_kernel(x_ref, w1_ref, b1_ref, w2_ref, b2_ref, w3_ref, b3_ref,
                  w4_ref, b4_ref, w5_ref, b5_ref, wf_ref, bf_ref, o_ref,
                  *, nc):
    xb = x_ref[...].astype(jnp.bfloat16)
    x = jnp.pad(xb, ((0, 0), (0, 0), (64, 64)))

    def mm(v, w, b):
        acc = jnp.dot(v, w, preferred_element_type=jnp.float32) + b
        return jnp.maximum(acc, 0.0).astype(jnp.bfloat16)

    b1 = b1_ref[...]
    rows = []
    for ho in range(8):
        acc = jnp.dot(x[:, 0, 128 * ho:128 * ho + 256], w1_ref[0],
                      preferred_element_type=jnp.float32)
        acc += jnp.dot(x[:, 1, 128 * ho:128 * ho + 256], w1_ref[1],
                       preferred_element_type=jnp.float32)
        acc += jnp.dot(x[:, 2, 128 * ho:128 * ho + 256], w1_ref[2],
                       preferred_element_type=jnp.float32)
        rows.append(jnp.maximum(acc + b1, 0.0).astype(jnp.bfloat16))

    p1 = []
    for h2 in range(4):
        h = jnp.maximum(rows[2 * h2], rows[2 * h2 + 1])
        p1 += [jnp.maximum(h[:, w * 256:w * 256 + 128],
                           h[:, w * 256 + 128:w * 256 + 256]) for w in range(4)]
    x2 = jnp.concatenate(p1, axis=1)

    b2 = b2_ref[...]
    r2 = [mm(x2, w2_ref[ho], b2) for ho in range(4)]

    p2 = []
    for h2 in range(2):
        h = jnp.maximum(r2[2 * h2], r2[2 * h2 + 1])
        p2 += [jnp.maximum(h[:, 0:256], h[:, 256:512]),
               jnp.maximum(h[:, 512:768], h[:, 768:1024])]
    x3 = jnp.concatenate(p2, axis=1)

    x4 = mm(x3, w3_ref[...], b3_ref[...])
    x5 = mm(x4, w4_ref[...], b4_ref[...])
    x6 = mm(x5, w5_ref[...], b5_ref[...])

    p3 = jnp.maximum(jnp.maximum(x6[:, 0:256], x6[:, 256:512]),
                     jnp.maximum(x6[:, 512:768], x6[:, 768:1024]))

    acc = jnp.dot(p3, wf_ref[...], preferred_element_type=jnp.float32)
    o_ref[...] = (acc + bf_ref[...])[:, :nc]


def kernel(x, conv1_w, conv1_b, conv2_w, conv2_b, conv3_w, conv3_b,
           conv4_w, conv4_b, conv5_w, conv5_b, fc_w, fc_b):
    import functools
    B = x.shape[0]
    xs = x.reshape(B, 3, 1024)

    w1d = _conv1_w(conv1_w).astype(jnp.bfloat16)
    b1d = jnp.tile(conv1_b, 8).reshape(1, 1024).astype(jnp.float32)

    w2s = _dense_conv_w(conv2_w, 4, 4, 128, 256, row_major_out=True)
    w2s = w2s.astype(jnp.bfloat16)
    b2d = jnp.tile(jnp.pad(conv2_b, (0, 64)), 4).reshape(1, 1024)
    b2d = b2d.astype(jnp.float32)

    w3d = _dense_conv_w(conv3_w, 2, 2, 256, 384).astype(jnp.bfloat16)
    b3d = jnp.tile(conv3_b, 4).reshape(1, 1536).astype(jnp.float32)
    w4d = _dense_conv_w(conv4_w, 2, 2, 384, 256).astype(jnp.bfloat16)
    b4d = jnp.tile(conv4_b, 4).reshape(1, 1024).astype(jnp.float32)
    w5d = _dense_conv_w(conv5_w, 2, 2, 256, 256).astype(jnp.bfloat16)
    b5d = jnp.tile(conv5_b, 4).reshape(1, 1024).astype(jnp.float32)

    NC = fc_w.shape[1]
    NCp = ((NC + 127) // 128) * 128
    wf = jnp.pad(fc_w, ((0, 0), (0, NCp - NC))).astype(jnp.bfloat16)
    bf = jnp.pad(fc_b, ((0, NCp - NC))).reshape(1, NCp).astype(jnp.float32)

    consts = (w1d, b1d, w2s, b2d, w3d, b3d, w4d, b4d, w5d, b5d, wf, bf)
    TB = 512
    tb = min(TB, B)
    Bp = ((B + tb - 1) // tb) * tb
    if Bp != B:
        xs = jnp.pad(xs, ((0, Bp - B), (0, 0), (0, 0)))
    out = pl.pallas_call(
        functools.partial(_fused_kernel, nc=NC),
        out_shape=jax.ShapeDtypeStruct((Bp, NC), jnp.float32),
        grid=(Bp // tb,),
        in_specs=[pl.BlockSpec((tb, 3, 1024), lambda i: (i, 0, 0))]
        + [pl.BlockSpec(c.shape, lambda i, _n=c.ndim: (0,) * _n) for c in consts],
        out_specs=pl.BlockSpec((tb, NC), lambda i: (i, 0)),
        compiler_params=pltpu.CompilerParams(dimension_semantics=("parallel",)),
    )(xs, *consts)
    return out[:B]


# speedup vs baseline: 1.0433x; 1.0433x over previous
import jax
import jax.numpy as jnp
from jax.experimental import pallas as pl
from jax.experimental.pallas import tpu as pltpu


def _dense_conv_w(w, hw_in, hw_out, c_in_pad, c_out_pad, row_major_out=False):
    kh, kw, ci, co = w.shape
    off, P = kh // 2, hw_out - 1
    wp = jnp.pad(w, ((P, P), (P, P), (0, c_in_pad - ci), (0, c_out_pad - co)))
    K = hw_in * hw_in * c_in_pad
    rows = []
    for ho in range(hw_out):
        hs = wp[off + P - ho:off + P - ho + hw_in]
        cols = [hs[:, off + P - wo:off + P - wo + hw_in] for wo in range(hw_out)]
        rows.append(jnp.stack(cols, axis=3))
    if row_major_out:
        return jnp.stack(rows, axis=0).reshape(hw_out, K, hw_out * c_out_pad)
    return jnp.stack(rows, axis=3).reshape(K, hw_out * hw_out * c_out_pad)


def _conv1_w(w):
    co = w.shape[-1]
    wt = w.transpose(2, 0, 1, 3)
    blocks = []
    for wo in range(8):
        lo = 4 * wo - 2
        s_lo, s_hi = max(0, -lo), min(7, 32 - lo)
        blk = jnp.pad(wt[:, :, s_lo:s_hi],
                      ((0, 0), (0, 1), (lo + s_lo, 32 - lo - s_hi), (0, 0)))
        blocks.append(blk)
    return jnp.stack(blocks, axis=3).reshape(3, 256, 8 * co)


def _fused_kernel(x_ref, w1_ref, b1_ref, w2_ref, b2_ref, w3_ref, b3_ref,
                  w4_ref, b4_ref, w5_ref, b5_ref, wf_ref, bf_ref, o_ref,
                  *, nc):
    xb = x_ref[...].astype(jnp.bfloat16)
    x = jnp.pad(xb, ((0, 0), (0, 0), (64, 64)))

    def mm(v, w, b):
        acc = jnp.dot(v, w, preferred_element_type=jnp.float32) + b
        return jnp.maximum(acc, 0.0).astype(jnp.bfloat16)

    b1 = b1_ref[...]
    rows = []
    for ho in range(8):
        acc = jnp.dot(x[:, 0, 128 * ho:128 * ho + 256], w1_ref[0],
                      preferred_element_type=jnp.float32)
        acc += jnp.dot(x[:, 1, 128 * ho:128 * ho + 256], w1_ref[1],
                       preferred_element_type=jnp.float32)
        acc += jnp.dot(x[:, 2, 128 * ho:128 * ho + 256], w1_ref[2],
                       preferred_element_type=jnp.float32)
        rows.append(jnp.maximum(acc + b1, 0.0).astype(jnp.bfloat16))

    p1 = []
    for h2 in range(4):
        h = jnp.maximum(rows[2 * h2], rows[2 * h2 + 1])
        p1 += [jnp.maximum(h[:, w * 256:w * 256 + 128],
                           h[:, w * 256 + 128:w * 256 + 256]) for w in range(4)]
    x2 = jnp.concatenate(p1, axis=1)

    b2 = b2_ref[...]
    r2 = [mm(x2[:, 0:1536], w2_ref[0, 0:1536, :], b2),
          mm(x2, w2_ref[1], b2),
          mm(x2, w2_ref[2], b2),
          mm(x2[:, 512:2048], w2_ref[3, 512:2048, :], b2)]

    p2 = []
    for h2 in range(2):
        h = jnp.maximum(r2[2 * h2], r2[2 * h2 + 1])
        p2 += [jnp.maximum(h[:, 0:256], h[:, 256:512]),
               jnp.maximum(h[:, 512:768], h[:, 768:1024])]
    x3 = jnp.concatenate(p2, axis=1)

    x4 = mm(x3, w3_ref[...], b3_ref[...])
    x5 = mm(x4, w4_ref[...], b4_ref[...])
    x6 = mm(x5, w5_ref[...], b5_ref[...])

    p3 = jnp.maximum(jnp.maximum(x6[:, 0:256], x6[:, 256:512]),
                     jnp.maximum(x6[:, 512:768], x6[:, 768:1024]))

    acc = jnp.dot(p3, wf_ref[...], preferred_element_type=jnp.float32)
    o_ref[...] = (acc + bf_ref[...])[:, :nc]


def kernel(x, conv1_w, conv1_b, conv2_w, conv2_b, conv3_w, conv3_b,
           conv4_w, conv4_b, conv5_w, conv5_b, fc_w, fc_b):
    import functools
    B = x.shape[0]
    xs = x.reshape(B, 3, 1024)

    w1d = _conv1_w(conv1_w).astype(jnp.bfloat16)
    b1d = jnp.tile(conv1_b, 8).reshape(1, 1024).astype(jnp.float32)

    w2s = _dense_conv_w(conv2_w, 4, 4, 128, 256, row_major_out=True)
    w2s = w2s.astype(jnp.bfloat16)
    b2d = jnp.tile(jnp.pad(conv2_b, (0, 64)), 4).reshape(1, 1024)
    b2d = b2d.astype(jnp.float32)

    w3d = _dense_conv_w(conv3_w, 2, 2, 256, 384).astype(jnp.bfloat16)
    b3d = jnp.tile(conv3_b, 4).reshape(1, 1536).astype(jnp.float32)
    w4d = _dense_conv_w(conv4_w, 2, 2, 384, 256).astype(jnp.bfloat16)
    b4d = jnp.tile(conv4_b, 4).reshape(1, 1024).astype(jnp.float32)
    w5d = _dense_conv_w(conv5_w, 2, 2, 256, 256).astype(jnp.bfloat16)
    b5d = jnp.tile(conv5_b, 4).reshape(1, 1024).astype(jnp.float32)

    NC = fc_w.shape[1]
    NCp = ((NC + 127) // 128) * 128
    wf = jnp.pad(fc_w, ((0, 0), (0, NCp - NC))).astype(jnp.bfloat16)
    bf = jnp.pad(fc_b, ((0, NCp - NC))).reshape(1, NCp).astype(jnp.float32)

    consts = (w1d, b1d, w2s, b2d, w3d, b3d, w4d, b4d, w5d, b5d, wf, bf)
    TB = 256
    tb = min(TB, B)
    Bp = ((B + tb - 1) // tb) * tb
    if Bp != B:
        xs = jnp.pad(xs, ((0, Bp - B), (0, 0), (0, 0)))
    out = pl.pallas_call(
        functools.partial(_fused_kernel, nc=NC),
        out_shape=jax.ShapeDtypeStruct((Bp, NC), jnp.float32),
        grid=(Bp // tb,),
        in_specs=[pl.BlockSpec((tb, 3, 1024), lambda i: (i, 0, 0))]
        + [pl.BlockSpec(c.shape, lambda i, _n=c.ndim: (0,) * _n) for c in consts],
        out_specs=pl.BlockSpec((tb, NC), lambda i: (i, 0)),
        compiler_params=pltpu.CompilerParams(dimension_semantics=("parallel",)),
    )(xs, *consts)
    return out[:B]
